# Initial kernel scaffold; baseline (speedup 1.0000x reference)
import jax
import jax.numpy as jnp
from jax.experimental import pallas as pl


def kernel(low_theta, low_phi, low_feats, high_theta, high_phi, high_feats, W0, b0, g0, be0, W1, b1, g1, be1):
    raise NotImplementedError("write your pallas kernel here")


# speedup vs baseline: 1.9305x; 1.9305x over previous
import functools

import jax
import jax.numpy as jnp
from jax import lax
from jax.experimental import pallas as pl
from jax.experimental.pallas import tpu as pltpu
from jax.experimental.pallas import tpu_sc as plsc

_B, _N, _M, _C1, _C2 = 8, 4096, 1024, 128, 256
_CH = 256
_ROWS = _B * _N
_EPS = 1e-5


_NBLK_IDX = 1024
_NB_IDX = _N // _NBLK_IDX


def _idx_body(lt_ref, lp_ref, ht_ref, hp_ref, idx_ref):
    b = pl.program_id(0)
    lt = lt_ref[0]
    lp = lp_ref[0]
    ht = ht_ref[0]
    hp = hp_ref[0]
    dt = lt - ht
    dp = lp - hp
    dist = jnp.sqrt(dt * dt + dp * dp)
    dmin = jnp.min(dist, axis=1, keepdims=True)
    iot = lax.broadcasted_iota(jnp.int32, (_NBLK_IDX, _M), 1)
    cand = jnp.where(dist == dmin, iot, _M)
    imin = jnp.min(cand, axis=1, keepdims=True)
    idx_ref[0] = imin + b * _M


def _compute_idx(low_theta, low_phi, high_theta, high_phi):
    lt3 = low_theta.reshape(_B, _N, 1)
    lp3 = low_phi.reshape(_B, _N, 1)
    ht3 = high_theta.reshape(_B, 1, _M)
    hp3 = high_phi.reshape(_B, 1, _M)
    idx = pl.pallas_call(
        _idx_body,
        grid=(_B, _NB_IDX),
        in_specs=[
            pl.BlockSpec((1, _NBLK_IDX, 1), lambda b, n: (b, n, 0)),
            pl.BlockSpec((1, _NBLK_IDX, 1), lambda b, n: (b, n, 0)),
            pl.BlockSpec((1, 1, _M), lambda b, n: (b, 0, 0)),
            pl.BlockSpec((1, 1, _M), lambda b, n: (b, 0, 0)),
        ],
        out_specs=pl.BlockSpec((1, _NBLK_IDX, 1), lambda b, n: (b, n, 0)),
        out_shape=jax.ShapeDtypeStruct((_B, _N, 1), jnp.int32),
    )(lt3, lp3, ht3, hp3)
    return idx.reshape(_ROWS)


def _qt_body(hf_ref, w_ref, qt_ref):
    hf = hf_ref[0]
    w = w_ref[...]
    qt_ref[0] = lax.dot_general(
        hf, w, (((0,), (0,)), ((), ())), preferred_element_type=jnp.float32
    )


def _compute_qt(high_feats, w0bt):
    return pl.pallas_call(
        _qt_body,
        grid=(_B,),
        in_specs=[
            pl.BlockSpec((1, _C2, _M), lambda b: (b, 0, 0)),
            pl.BlockSpec((_C2, _CH), lambda b: (0, 0)),
        ],
        out_specs=pl.BlockSpec((1, _M, _CH), lambda b: (b, 0, 0)),
        out_shape=jax.ShapeDtypeStruct((_B, _M, _CH), jnp.float32),
    )(high_feats, w0bt)


_NC = 2
_NS = 16
_NW = _NC * _NS
_RPW = _ROWS // _NW
_CHUNK = 128
_NCHUNK = _RPW // _CHUNK


def _sc_gather_body(table_hbm, idx_hbm, out_hbm, idx_v, buf0, buf1, sem0, sem1):
    wid = lax.axis_index("s") * _NC + lax.axis_index("c")
    base = wid * _RPW
    pltpu.sync_copy(idx_hbm.at[pl.ds(base, _RPW)], idx_v)
    bufs = (buf0, buf1)
    sems = (sem0, sem1)
    copies = [None, None]
    for j in range(_NCHUNK):
        p = j % 2
        if copies[p] is not None:
            copies[p].wait()
            pltpu.sync_copy(bufs[p], out_hbm.at[pl.ds(base + (j - 2) * _CHUNK, _CHUNK)])
        copies[p] = pltpu.async_copy(
            table_hbm.at[idx_v.at[pl.ds(j * _CHUNK, _CHUNK)]], bufs[p], sems[p]
        )
    for j in range(_NCHUNK - 2, _NCHUNK):
        p = j % 2
        copies[p].wait()
        pltpu.sync_copy(bufs[p], out_hbm.at[pl.ds(base + j * _CHUNK, _CHUNK)])


def _sc_gather(table, idx):
    k = functools.partial(
        pl.kernel,
        out_type=jax.ShapeDtypeStruct((_ROWS, _CH), jnp.float32),
        mesh=plsc.VectorSubcoreMesh(core_axis_name="c", subcore_axis_name="s"),
        scratch_types=[
            pltpu.VMEM((_RPW,), jnp.int32),
            pltpu.VMEM((_CHUNK, _CH), jnp.float32),
            pltpu.VMEM((_CHUNK, _CH), jnp.float32),
            pltpu.SemaphoreType.DMA,
            pltpu.SemaphoreType.DMA,
        ],
    )(_sc_gather_body)
    return k(table, idx)


_NBLK_L = 512
_NB_L = _N // _NBLK_L


def _l0_body(lf_ref, g_ref, w_ref, h0_ref, st_ref):
    @pl.when(jnp.logical_and(pl.program_id(0) == 0, pl.program_id(1) == 0))
    def _():
        st_ref[...] = jnp.zeros_like(st_ref)

    lf = lf_ref[0]
    g = g_ref[0]
    w = w_ref[...]
    h = (
        lax.dot_general(lf, w, (((0,), (0,)), ((), ())), preferred_element_type=jnp.float32)
        + g
    )
    h0_ref[0] = h
    st_ref[0:1, :] += jnp.sum(h, axis=0, keepdims=True)
    st_ref[1:2, :] += jnp.sum(h * h, axis=0, keepdims=True)


def _layer0(low_feats, g_rows, w0at):
    return pl.pallas_call(
        _l0_body,
        grid=(_B, _NB_L),
        in_specs=[
            pl.BlockSpec((1, _C1, _NBLK_L), lambda b, n: (b, 0, n)),
            pl.BlockSpec((1, _NBLK_L, _CH), lambda b, n: (b, n, 0)),
            pl.BlockSpec((_C1, _CH), lambda b, n: (0, 0)),
        ],
        out_specs=[
            pl.BlockSpec((1, _NBLK_L, _CH), lambda b, n: (b, n, 0)),
            pl.BlockSpec((8, _CH), lambda b, n: (0, 0)),
        ],
        out_shape=[
            jax.ShapeDtypeStruct((_B, _N, _CH), jnp.float32),
            jax.ShapeDtypeStruct((8, _CH), jnp.float32),
        ],
    )(low_feats, g_rows, w0at)


def _l1_body(h0_ref, sc_ref, sh_ref, w_ref, h1_ref, st_ref):
    @pl.when(jnp.logical_and(pl.program_id(0) == 0, pl.program_id(1) == 0))
    def _():
        st_ref[...] = jnp.zeros_like(st_ref)

    h0 = h0_ref[0]
    x = jnp.maximum(h0 * sc_ref[...] + sh_ref[...], 0.0)
    h1 = lax.dot_general(
        w_ref[...], x, (((1,), (1,)), ((), ())), preferred_element_type=jnp.float32
    )
    h1_ref[0] = h1
    st_ref[:, 0:1] += jnp.sum(h1, axis=1, keepdims=True)
    st_ref[:, 1:2] += jnp.sum(h1 * h1, axis=1, keepdims=True)


def _layer1(h0, scale0, shift0, w1):
    return pl.pallas_call(
        _l1_body,
        grid=(_B, _NB_L),
        in_specs=[
            pl.BlockSpec((1, _NBLK_L, _CH), lambda b, n: (b, n, 0)),
            pl.BlockSpec((1, _CH), lambda b, n: (0, 0)),
            pl.BlockSpec((1, _CH), lambda b, n: (0, 0)),
            pl.BlockSpec((_CH, _CH), lambda b, n: (0, 0)),
        ],
        out_specs=[
            pl.BlockSpec((1, _CH, _NBLK_L), lambda b, n: (b, 0, n)),
            pl.BlockSpec((_CH, 8), lambda b, n: (0, 0)),
        ],
        out_shape=[
            jax.ShapeDtypeStruct((_B, _CH, _N), jnp.float32),
            jax.ShapeDtypeStruct((_CH, 8), jnp.float32),
        ],
    )(h0, scale0, shift0, w1)


def _fin_body(h1_ref, sc_ref, sh_ref, o_ref):
    o_ref[0] = jnp.maximum(h1_ref[0] * sc_ref[...] + sh_ref[...], 0.0)


def _finalize(h1, scale1, shift1):
    return pl.pallas_call(
        _fin_body,
        grid=(_B, _NB_L),
        in_specs=[
            pl.BlockSpec((1, _CH, _NBLK_L), lambda b, n: (b, 0, n)),
            pl.BlockSpec((_CH, 1), lambda b, n: (0, 0)),
            pl.BlockSpec((_CH, 1), lambda b, n: (0, 0)),
        ],
        out_specs=pl.BlockSpec((1, _CH, _NBLK_L), lambda b, n: (b, 0, n)),
        out_shape=jax.ShapeDtypeStruct((_B, _CH, _N), jnp.float32),
    )(h1, scale1, shift1)


def kernel(low_theta, low_phi, low_feats, high_theta, high_phi, high_feats,
           W0, b0, g0, be0, W1, b1, g1, be1):
    del b0, b1
    w0at = W0[:, :_C1].T
    w0bt = W0[:, _C1:].T

    idx = _compute_idx(low_theta, low_phi, high_theta, high_phi)
    qt = _compute_qt(high_feats, w0bt).reshape(_B * _M, _CH)
    g_rows = _sc_gather(qt, idx).reshape(_B, _N, _CH)
    h0, st0 = _layer0(low_feats, g_rows, w0at)

    cnt = float(_ROWS)
    mean0 = st0[0] / cnt
    var0 = st0[1] / cnt - mean0 * mean0
    scale0 = (g0 / jnp.sqrt(var0 + _EPS)).reshape(1, _CH)
    shift0 = (be0 - scale0[0] * mean0).reshape(1, _CH)

    h1, st1 = _layer1(h0, scale0, shift0, W1)

    mean1 = st1[:, 0] / cnt
    var1 = st1[:, 1] / cnt - mean1 * mean1
    scale1 = (g1 / jnp.sqrt(var1 + _EPS)).reshape(_CH, 1)
    shift1 = (be1 - scale1[:, 0] * mean1).reshape(_CH, 1)

    return _finalize(h1, scale1, shift1)


# speedup vs baseline: 2.1190x; 1.0976x over previous
import functools

import jax
import jax.numpy as jnp
from jax import lax
from jax.experimental import pallas as pl
from jax.experimental.pallas import tpu as pltpu
from jax.experimental.pallas import tpu_sc as plsc

_B, _N, _M, _C1, _C2 = 8, 4096, 1024, 128, 256
_CH = 256
_ROWS = _B * _N
_CNT = float(_ROWS)
_EPS = 1e-5


_NBLK_IDX = 1024
_NB_IDX = _N // _NBLK_IDX


def _idx_qt_body(lt_ref, lp_ref, ht_ref, hp_ref, hf_ref, w_ref, idx_ref, qt_ref):
    b = pl.program_id(0)

    @pl.when(pl.program_id(1) == 0)
    def _():
        qt_ref[0] = lax.dot_general(
            hf_ref[0], w_ref[...], (((0,), (0,)), ((), ())),
            preferred_element_type=jnp.float32,
        )

    lt = lt_ref[0]
    lp = lp_ref[0]
    ht = ht_ref[0]
    hp = hp_ref[0]
    dt = lt - ht
    dp = lp - hp
    dist = dt * dt + dp * dp
    imin = jnp.argmin(dist, axis=1).astype(jnp.int32).reshape(_NBLK_IDX, 1)
    idx_ref[0] = imin + b * _M


def _idx_qt(low_theta, low_phi, high_theta, high_phi, high_feats, w0bt):
    lt3 = low_theta.reshape(_B, _N, 1)
    lp3 = low_phi.reshape(_B, _N, 1)
    ht3 = high_theta.reshape(_B, 1, _M)
    hp3 = high_phi.reshape(_B, 1, _M)
    idx, qt = pl.pallas_call(
        _idx_qt_body,
        grid=(_B, _NB_IDX),
        in_specs=[
            pl.BlockSpec((1, _NBLK_IDX, 1), lambda b, n: (b, n, 0)),
            pl.BlockSpec((1, _NBLK_IDX, 1), lambda b, n: (b, n, 0)),
            pl.BlockSpec((1, 1, _M), lambda b, n: (b, 0, 0)),
            pl.BlockSpec((1, 1, _M), lambda b, n: (b, 0, 0)),
            pl.BlockSpec((1, _C2, _M), lambda b, n: (b, 0, 0)),
            pl.BlockSpec((_C2, _CH), lambda b, n: (0, 0)),
        ],
        out_specs=[
            pl.BlockSpec((1, _NBLK_IDX, 1), lambda b, n: (b, n, 0)),
            pl.BlockSpec((1, _M, _CH), lambda b, n: (b, 0, 0)),
        ],
        out_shape=[
            jax.ShapeDtypeStruct((_B, _N, 1), jnp.int32),
            jax.ShapeDtypeStruct((_B, _M, _CH), jnp.float32),
        ],
    )(lt3, lp3, ht3, hp3, high_feats, w0bt)
    return idx.reshape(_ROWS), qt.reshape(_B * _M, _CH)


_NC = 2
_NS = 16
_NW = _NC * _NS
_RPW = _ROWS // _NW
_CHUNK = 128
_NCHUNK = _RPW // _CHUNK


def _sc_gather_body(table_hbm, idx_hbm, out_hbm, idx_v, buf0, buf1, sem0, sem1):
    wid = lax.axis_index("s") * _NC + lax.axis_index("c")
    base = wid * _RPW
    pltpu.sync_copy(idx_hbm.at[pl.ds(base, _RPW)], idx_v)
    bufs = (buf0, buf1)
    sems = (sem0, sem1)
    copies = [None, None]
    for j in range(_NCHUNK):
        p = j % 2
        if copies[p] is not None:
            copies[p].wait()
            pltpu.sync_copy(bufs[p], out_hbm.at[pl.ds(base + (j - 2) * _CHUNK, _CHUNK)])
        copies[p] = pltpu.async_copy(
            table_hbm.at[idx_v.at[pl.ds(j * _CHUNK, _CHUNK)]], bufs[p], sems[p]
        )
    for j in range(_NCHUNK - 2, _NCHUNK):
        p = j % 2
        copies[p].wait()
        pltpu.sync_copy(bufs[p], out_hbm.at[pl.ds(base + j * _CHUNK, _CHUNK)])


def _sc_gather(table, idx):
    k = functools.partial(
        pl.kernel,
        out_type=jax.ShapeDtypeStruct((_ROWS, _CH), jnp.float32),
        mesh=plsc.VectorSubcoreMesh(core_axis_name="c", subcore_axis_name="s"),
        scratch_types=[
            pltpu.VMEM((_RPW,), jnp.int32),
            pltpu.VMEM((_CHUNK, _CH), jnp.float32),
            pltpu.VMEM((_CHUNK, _CH), jnp.float32),
            pltpu.SemaphoreType.DMA,
            pltpu.SemaphoreType.DMA,
        ],
    )(_sc_gather_body)
    return k(table, idx)


_NBLK_L = 512
_NB_L = _N // _NBLK_L


def _l0_body(lf_ref, g_ref, w_ref, h0_ref, st_ref):
    @pl.when(jnp.logical_and(pl.program_id(0) == 0, pl.program_id(1) == 0))
    def _():
        st_ref[...] = jnp.zeros_like(st_ref)

    lf = lf_ref[0]
    g = g_ref[0]
    w = w_ref[...]
    h = (
        lax.dot_general(lf, w, (((0,), (0,)), ((), ())), preferred_element_type=jnp.float32)
        + g
    )
    h0_ref[0] = h
    st_ref[0:1, :] += jnp.sum(h, axis=0, keepdims=True)
    st_ref[1:2, :] += jnp.sum(h * h, axis=0, keepdims=True)


def _layer0(low_feats, g_rows, w0at):
    return pl.pallas_call(
        _l0_body,
        grid=(_B, _NB_L),
        in_specs=[
            pl.BlockSpec((1, _C1, _NBLK_L), lambda b, n: (b, 0, n)),
            pl.BlockSpec((1, _NBLK_L, _CH), lambda b, n: (b, n, 0)),
            pl.BlockSpec((_C1, _CH), lambda b, n: (0, 0)),
        ],
        out_specs=[
            pl.BlockSpec((1, _NBLK_L, _CH), lambda b, n: (b, n, 0)),
            pl.BlockSpec((8, _CH), lambda b, n: (0, 0)),
        ],
        out_shape=[
            jax.ShapeDtypeStruct((_B, _N, _CH), jnp.float32),
            jax.ShapeDtypeStruct((8, _CH), jnp.float32),
        ],
    )(low_feats, g_rows, w0at)


def _l1_body(h0_ref, sc_ref, sh_ref, w_ref, g1_ref, be1_ref, o_ref, st_ref):
    p = pl.program_id(0)

    @pl.when(jnp.logical_and(p == 0,
                             jnp.logical_and(pl.program_id(1) == 0,
                                             pl.program_id(2) == 0)))
    def _():
        st_ref[...] = jnp.zeros_like(st_ref)

    h0 = h0_ref[0]
    x = jnp.maximum(h0 * sc_ref[...] + sh_ref[...], 0.0)
    h1 = lax.dot_general(
        w_ref[...], x, (((1,), (1,)), ((), ())), preferred_element_type=jnp.float32
    )

    @pl.when(p == 0)
    def _():
        st_ref[:, 0:1] += jnp.sum(h1, axis=1, keepdims=True)
        st_ref[:, 1:2] += jnp.sum(h1 * h1, axis=1, keepdims=True)

    @pl.when(p == 1)
    def _():
        mean1 = st_ref[:, 0:1] * (1.0 / _CNT)
        var1 = st_ref[:, 1:2] * (1.0 / _CNT) - mean1 * mean1
        scale1 = g1_ref[...] / jnp.sqrt(var1 + _EPS)
        shift1 = be1_ref[...] - scale1 * mean1
        o_ref[0] = jnp.maximum(h1 * scale1 + shift1, 0.0)


def _layer1_fin(h0, scale0, shift0, w1, g1, be1):
    out, _ = pl.pallas_call(
        _l1_body,
        grid=(2, _B, _NB_L),
        in_specs=[
            pl.BlockSpec((1, _NBLK_L, _CH), lambda p, b, n: (b, n, 0)),
            pl.BlockSpec((1, _CH), lambda p, b, n: (0, 0)),
            pl.BlockSpec((1, _CH), lambda p, b, n: (0, 0)),
            pl.BlockSpec((_CH, _CH), lambda p, b, n: (0, 0)),
            pl.BlockSpec((_CH, 1), lambda p, b, n: (0, 0)),
            pl.BlockSpec((_CH, 1), lambda p, b, n: (0, 0)),
        ],
        out_specs=[
            pl.BlockSpec((1, _CH, _NBLK_L), lambda p, b, n: (b * p, 0, n * p)),
            pl.BlockSpec((_CH, 8), lambda p, b, n: (0, 0)),
        ],
        out_shape=[
            jax.ShapeDtypeStruct((_B, _CH, _N), jnp.float32),
            jax.ShapeDtypeStruct((_CH, 8), jnp.float32),
        ],
    )(h0, scale0, shift0, w1, g1, be1)
    return out


def kernel(low_theta, low_phi, low_feats, high_theta, high_phi, high_feats,
           W0, b0, g0, be0, W1, b1, g1, be1):
    del b0, b1
    w0at = W0[:, :_C1].T
    w0bt = W0[:, _C1:].T

    idx, qt = _idx_qt(low_theta, low_phi, high_theta, high_phi, high_feats, w0bt)
    g_rows = _sc_gather(qt, idx).reshape(_B, _N, _CH)
    h0, st0 = _layer0(low_feats, g_rows, w0at)

    mean0 = st0[0] / _CNT
    var0 = st0[1] / _CNT - mean0 * mean0
    scale0 = (g0 / jnp.sqrt(var0 + _EPS)).reshape(1, _CH)
    shift0 = (be0 - scale0[0] * mean0).reshape(1, _CH)

    return _layer1_fin(h0, scale0, shift0, W1,
                       g1.reshape(_CH, 1), be1.reshape(_CH, 1))


# speedup vs baseline: 2.4948x; 1.1773x over previous
import functools

import jax
import jax.numpy as jnp
from jax import lax
from jax.experimental import pallas as pl
from jax.experimental.pallas import tpu as pltpu
from jax.experimental.pallas import tpu_sc as plsc

_B, _N, _M, _C1, _C2 = 8, 4096, 1024, 128, 256
_CH = 256
_ROWS = _B * _N
_CNT = float(_ROWS)
_EPS = 1e-5


_NBLK_IDX = 1024
_NB_IDX = _N // _NBLK_IDX


def _idx_qt_body(lt_ref, lp_ref, ht_ref, hp_ref, hf_ref, w_ref, idx_ref, qt_ref):
    b = pl.program_id(0)

    @pl.when(pl.program_id(1) == 0)
    def _():
        qt_ref[...] = lax.dot_general(
            hf_ref[0], w_ref[...], (((0,), (0,)), ((), ())),
            preferred_element_type=jnp.float32,
        )

    lt = lt_ref[0]
    lp = lp_ref[0]
    ht = ht_ref[0]
    hp = hp_ref[0]
    dt = lt - ht
    dp = lp - hp
    dist = dt * dt + dp * dp
    imin = jnp.argmin(dist, axis=1).astype(jnp.int32).reshape(_NBLK_IDX, 1)
    idx_ref[0] = imin + b * _M


def _idx_qt(low_theta, low_phi, high_theta, high_phi, high_feats, w0bt):
    lt3 = low_theta.reshape(_B, _N, 1)
    lp3 = low_phi.reshape(_B, _N, 1)
    ht3 = high_theta.reshape(_B, 1, _M)
    hp3 = high_phi.reshape(_B, 1, _M)
    idx, qt = pl.pallas_call(
        _idx_qt_body,
        grid=(_B, _NB_IDX),
        in_specs=[
            pl.BlockSpec((1, _NBLK_IDX, 1), lambda b, n: (b, n, 0)),
            pl.BlockSpec((1, _NBLK_IDX, 1), lambda b, n: (b, n, 0)),
            pl.BlockSpec((1, 1, _M), lambda b, n: (b, 0, 0)),
            pl.BlockSpec((1, 1, _M), lambda b, n: (b, 0, 0)),
            pl.BlockSpec((1, _C2, _M), lambda b, n: (b, 0, 0)),
            pl.BlockSpec((_C2, _CH), lambda b, n: (0, 0)),
        ],
        out_specs=[
            pl.BlockSpec((1, _NBLK_IDX, 1), lambda b, n: (b, n, 0)),
            pl.BlockSpec((_M, _CH), lambda b, n: (b, 0)),
        ],
        out_shape=[
            jax.ShapeDtypeStruct((_B, _N, 1), jnp.int32),
            jax.ShapeDtypeStruct((_B * _M, _CH), jnp.float32),
        ],
    )(lt3, lp3, ht3, hp3, high_feats, w0bt)
    return idx.reshape(_ROWS), qt


_NC = 2
_NS = 16
_NW = _NC * _NS
_RPW = _ROWS // _NW
_CHUNK = 128
_NCHUNK = _RPW // _CHUNK


def _sc_gather_body(table_hbm, idx_hbm, out_hbm, idx_v, buf0, buf1, sem0, sem1):
    wid = lax.axis_index("s") * _NC + lax.axis_index("c")
    base = wid * _RPW
    pltpu.sync_copy(idx_hbm.at[pl.ds(base, _RPW)], idx_v)
    bufs = (buf0, buf1)
    sems = (sem0, sem1)
    copies = [None, None]
    for j in range(_NCHUNK):
        p = j % 2
        if copies[p] is not None:
            copies[p].wait()
            pltpu.sync_copy(bufs[p], out_hbm.at[pl.ds(base + (j - 2) * _CHUNK, _CHUNK)])
        copies[p] = pltpu.async_copy(
            table_hbm.at[idx_v.at[pl.ds(j * _CHUNK, _CHUNK)]], bufs[p], sems[p]
        )
    for j in range(_NCHUNK - 2, _NCHUNK):
        p = j % 2
        copies[p].wait()
        pltpu.sync_copy(bufs[p], out_hbm.at[pl.ds(base + j * _CHUNK, _CHUNK)])


def _sc_gather(table, idx):
    k = functools.partial(
        pl.kernel,
        out_type=jax.ShapeDtypeStruct((_ROWS, _CH), jnp.float32),
        mesh=plsc.VectorSubcoreMesh(core_axis_name="c", subcore_axis_name="s"),
        scratch_types=[
            pltpu.VMEM((_RPW,), jnp.int32),
            pltpu.VMEM((_CHUNK, _CH), jnp.float32),
            pltpu.VMEM((_CHUNK, _CH), jnp.float32),
            pltpu.SemaphoreType.DMA,
            pltpu.SemaphoreType.DMA,
        ],
    )(_sc_gather_body)
    return k(table, idx)


_NBLK_L = 512
_NB_L = _N // _NBLK_L


def _dense_body(lf_ref, g_ref, w0_ref, w1_ref, g0_ref, be0_ref, g1_ref, be1_ref,
                o_ref, h0_s, st0_s, st1_s):
    p = pl.program_id(0)
    b = pl.program_id(1)
    n = pl.program_id(2)
    step = b * _NB_L + n
    first = jnp.logical_and(b == 0, n == 0)
    rows = pl.ds(step * _NBLK_L, _NBLK_L)

    @pl.when(p == 0)
    def _():
        @pl.when(first)
        def _():
            st0_s[...] = jnp.zeros_like(st0_s)

        h = (
            lax.dot_general(lf_ref[0], w0_ref[...], (((0,), (0,)), ((), ())),
                            preferred_element_type=jnp.float32)
            + g_ref[...]
        )
        h0_s[rows, :] = h
        st0_s[0:1, :] += jnp.sum(h, axis=0, keepdims=True)
        st0_s[1:2, :] += jnp.sum(h * h, axis=0, keepdims=True)

    def _bn0_relu():
        mean0 = st0_s[0:1, :] * (1.0 / _CNT)
        var0 = st0_s[1:2, :] * (1.0 / _CNT) - mean0 * mean0
        scale0 = g0_ref[...] / jnp.sqrt(var0 + _EPS)
        shift0 = be0_ref[...] - scale0 * mean0
        return jnp.maximum(h0_s[rows, :] * scale0 + shift0, 0.0)

    @pl.when(p == 1)
    def _():
        @pl.when(first)
        def _():
            st1_s[...] = jnp.zeros_like(st1_s)

        h1 = lax.dot_general(w1_ref[...], _bn0_relu(), (((1,), (1,)), ((), ())),
                             preferred_element_type=jnp.float32)
        st1_s[:, 0:1] += jnp.sum(h1, axis=1, keepdims=True)
        st1_s[:, 1:2] += jnp.sum(h1 * h1, axis=1, keepdims=True)

    @pl.when(p == 2)
    def _():
        h1 = lax.dot_general(w1_ref[...], _bn0_relu(), (((1,), (1,)), ((), ())),
                             preferred_element_type=jnp.float32)
        mean1 = st1_s[:, 0:1] * (1.0 / _CNT)
        var1 = st1_s[:, 1:2] * (1.0 / _CNT) - mean1 * mean1
        scale1 = g1_ref[...] / jnp.sqrt(var1 + _EPS)
        shift1 = be1_ref[...] - scale1 * mean1
        o_ref[0] = jnp.maximum(h1 * scale1 + shift1, 0.0)


def _dense(low_feats, g_rows, w0at, w1, g0, be0, g1, be1):
    def _p0(i):
        return i

    return pl.pallas_call(
        _dense_body,
        grid=(3, _B, _NB_L),
        in_specs=[
            pl.BlockSpec((1, _C1, _NBLK_L),
                         lambda p, b, n: (jnp.where(p == 0, b, 0), 0,
                                          jnp.where(p == 0, n, 0))),
            pl.BlockSpec((_NBLK_L, _CH),
                         lambda p, b, n: (jnp.where(p == 0, b * _NB_L + n, 0), 0)),
            pl.BlockSpec((_C1, _CH), lambda p, b, n: (0, 0)),
            pl.BlockSpec((_CH, _CH), lambda p, b, n: (0, 0)),
            pl.BlockSpec((1, _CH), lambda p, b, n: (0, 0)),
            pl.BlockSpec((1, _CH), lambda p, b, n: (0, 0)),
            pl.BlockSpec((_CH, 1), lambda p, b, n: (0, 0)),
            pl.BlockSpec((_CH, 1), lambda p, b, n: (0, 0)),
        ],
        out_specs=pl.BlockSpec(
            (1, _CH, _NBLK_L),
            lambda p, b, n: (jnp.where(p == 2, b, 0), 0,
                             jnp.where(p == 2, n, 0))),
        out_shape=jax.ShapeDtypeStruct((_B, _CH, _N), jnp.float32),
        scratch_shapes=[
            pltpu.VMEM((_ROWS, _CH), jnp.float32),
            pltpu.VMEM((8, _CH), jnp.float32),
            pltpu.VMEM((_CH, 8), jnp.float32),
        ],
    )(low_feats, g_rows, w0at, w1, g0, be0, g1, be1)


def kernel(low_theta, low_phi, low_feats, high_theta, high_phi, high_feats,
           W0, b0, g0, be0, W1, b1, g1, be1):
    del b0, b1
    w0at = W0[:, :_C1].T
    w0bt = W0[:, _C1:].T

    idx, qt = _idx_qt(low_theta, low_phi, high_theta, high_phi, high_feats, w0bt)
    g_rows = _sc_gather(qt, idx)
    return _dense(low_feats, g_rows, w0at, W1,
                  g0.reshape(1, _CH), be0.reshape(1, _CH),
                  g1.reshape(_CH, 1), be1.reshape(_CH, 1))


# speedup vs baseline: 2.6909x; 1.0786x over previous
import functools

import jax
import jax.numpy as jnp
from jax import lax
from jax.experimental import pallas as pl
from jax.experimental.pallas import tpu as pltpu
from jax.experimental.pallas import tpu_sc as plsc

_B, _N, _M, _C1, _C2 = 8, 4096, 1024, 128, 256
_CH = 256
_ROWS = _B * _N
_CNT = float(_ROWS)
_EPS = 1e-5


_NCHUNK_IDX = 4
_NBLK_IDX = _N // _NCHUNK_IDX


def _idx_qt_body(lt_ref, lp_ref, ht_ref, hp_ref, hf_ref, w_ref, idx_ref, qt_ref):
    b = pl.program_id(0)
    qt_ref[...] = lax.dot_general(
        hf_ref[0], w_ref[...], (((0,), (0,)), ((), ())),
        preferred_element_type=jnp.float32,
    )
    ht = jnp.transpose(ht_ref[0], (1, 0))
    hp = jnp.transpose(hp_ref[0], (1, 0))
    for c in range(_NCHUNK_IDX):
        sl = pl.ds(c * _NBLK_IDX, _NBLK_IDX)
        lt = lt_ref[0, 0:1, sl]
        lp = lp_ref[0, 0:1, sl]
        dt = ht - lt
        dp = hp - lp
        dist = dt * dt + dp * dp
        imin = jnp.argmin(dist, axis=0).astype(jnp.int32)
        idx_ref[0, 0, sl] = imin + b * _M


def _idx_qt(low_theta, low_phi, high_theta, high_phi, high_feats, w0bt):
    idx, qt = pl.pallas_call(
        _idx_qt_body,
        grid=(_B,),
        in_specs=[
            pl.BlockSpec((1, 1, _N), lambda b: (b, 0, 0)),
            pl.BlockSpec((1, 1, _N), lambda b: (b, 0, 0)),
            pl.BlockSpec((1, 1, _M), lambda b: (b, 0, 0)),
            pl.BlockSpec((1, 1, _M), lambda b: (b, 0, 0)),
            pl.BlockSpec((1, _C2, _M), lambda b: (b, 0, 0)),
            pl.BlockSpec((_C2, _CH), lambda b: (0, 0)),
        ],
        out_specs=[
            pl.BlockSpec((1, 1, _N), lambda b: (b, 0, 0)),
            pl.BlockSpec((_M, _CH), lambda b: (b, 0)),
        ],
        out_shape=[
            jax.ShapeDtypeStruct((_B, 1, _N), jnp.int32),
            jax.ShapeDtypeStruct((_B * _M, _CH), jnp.float32),
        ],
    )(low_theta.reshape(_B, 1, _N), low_phi.reshape(_B, 1, _N),
      high_theta.reshape(_B, 1, _M), high_phi.reshape(_B, 1, _M),
      high_feats, w0bt)
    return idx.reshape(_ROWS), qt


_NC = 2
_NS = 16
_NW = _NC * _NS
_RPW = _ROWS // _NW
_CHUNK = 128
_NCHUNK = _RPW // _CHUNK


def _sc_gather_body(table_hbm, idx_hbm, out_hbm, idx_v, buf0, buf1, sem0, sem1):
    wid = lax.axis_index("s") * _NC + lax.axis_index("c")
    base = wid * _RPW
    pltpu.sync_copy(idx_hbm.at[pl.ds(base, _RPW)], idx_v)
    bufs = (buf0, buf1)
    sems = (sem0, sem1)
    copies = [None, None]
    for j in range(_NCHUNK):
        p = j % 2
        if copies[p] is not None:
            copies[p].wait()
            pltpu.sync_copy(bufs[p], out_hbm.at[pl.ds(base + (j - 2) * _CHUNK, _CHUNK)])
        copies[p] = pltpu.async_copy(
            table_hbm.at[idx_v.at[pl.ds(j * _CHUNK, _CHUNK)]], bufs[p], sems[p]
        )
    for j in range(_NCHUNK - 2, _NCHUNK):
        p = j % 2
        copies[p].wait()
        pltpu.sync_copy(bufs[p], out_hbm.at[pl.ds(base + j * _CHUNK, _CHUNK)])


def _sc_gather(table, idx):
    k = functools.partial(
        pl.kernel,
        out_type=jax.ShapeDtypeStruct((_ROWS, _CH), jnp.float32),
        mesh=plsc.VectorSubcoreMesh(core_axis_name="c", subcore_axis_name="s"),
        scratch_types=[
            pltpu.VMEM((_RPW,), jnp.int32),
            pltpu.VMEM((_CHUNK, _CH), jnp.float32),
            pltpu.VMEM((_CHUNK, _CH), jnp.float32),
            pltpu.SemaphoreType.DMA,
            pltpu.SemaphoreType.DMA,
        ],
    )(_sc_gather_body)
    return k(table, idx)


_NBLK_L = 512
_NB_L = _N // _NBLK_L


def _dense_body(lf_ref, g_ref, w0_ref, w1_ref, g0_ref, be0_ref, g1_ref, be1_ref,
                o_ref, h0_s, st0_s, st1_s):
    p = pl.program_id(0)
    b = pl.program_id(1)
    n = pl.program_id(2)
    step = b * _NB_L + n
    first = jnp.logical_and(b == 0, n == 0)
    rows = pl.ds(step * _NBLK_L, _NBLK_L)

    @pl.when(p == 0)
    def _():
        @pl.when(first)
        def _():
            st0_s[...] = jnp.zeros_like(st0_s)

        h = (
            lax.dot_general(lf_ref[0], w0_ref[...], (((0,), (0,)), ((), ())),
                            preferred_element_type=jnp.float32)
            + g_ref[...]
        )
        h0_s[rows, :] = h
        st0_s[0:1, :] += jnp.sum(h, axis=0, keepdims=True)
        st0_s[1:2, :] += jnp.sum(h * h, axis=0, keepdims=True)

    def _bn0_relu():
        mean0 = st0_s[0:1, :] * (1.0 / _CNT)
        var0 = st0_s[1:2, :] * (1.0 / _CNT) - mean0 * mean0
        scale0 = g0_ref[...] / jnp.sqrt(var0 + _EPS)
        shift0 = be0_ref[...] - scale0 * mean0
        return jnp.maximum(h0_s[rows, :] * scale0 + shift0, 0.0)

    @pl.when(p == 1)
    def _():
        @pl.when(first)
        def _():
            st1_s[...] = jnp.zeros_like(st1_s)

        h1 = lax.dot_general(w1_ref[...], _bn0_relu(), (((1,), (1,)), ((), ())),
                             preferred_element_type=jnp.float32)
        st1_s[:, 0:1] += jnp.sum(h1, axis=1, keepdims=True)
        st1_s[:, 1:2] += jnp.sum(h1 * h1, axis=1, keepdims=True)

    @pl.when(p == 2)
    def _():
        h1 = lax.dot_general(w1_ref[...], _bn0_relu(), (((1,), (1,)), ((), ())),
                             preferred_element_type=jnp.float32)
        mean1 = st1_s[:, 0:1] * (1.0 / _CNT)
        var1 = st1_s[:, 1:2] * (1.0 / _CNT) - mean1 * mean1
        g1c = jnp.transpose(g1_ref[...], (1, 0))
        be1c = jnp.transpose(be1_ref[...], (1, 0))
        scale1 = g1c / jnp.sqrt(var1 + _EPS)
        shift1 = be1c - scale1 * mean1
        o_ref[0] = jnp.maximum(h1 * scale1 + shift1, 0.0)


def _dense(low_feats, g_rows, w0at, w1, g0, be0, g1, be1):
    def _p0(i):
        return i

    return pl.pallas_call(
        _dense_body,
        grid=(3, _B, _NB_L),
        in_specs=[
            pl.BlockSpec((1, _C1, _NBLK_L),
                         lambda p, b, n: (jnp.where(p == 0, b, 0), 0,
                                          jnp.where(p == 0, n, 0))),
            pl.BlockSpec((_NBLK_L, _CH),
                         lambda p, b, n: (jnp.where(p == 0, b * _NB_L + n, 0), 0)),
            pl.BlockSpec((_C1, _CH), lambda p, b, n: (0, 0)),
            pl.BlockSpec((_CH, _CH), lambda p, b, n: (0, 0)),
            pl.BlockSpec((1, _CH), lambda p, b, n: (0, 0)),
            pl.BlockSpec((1, _CH), lambda p, b, n: (0, 0)),
            pl.BlockSpec((1, _CH), lambda p, b, n: (0, 0)),
            pl.BlockSpec((1, _CH), lambda p, b, n: (0, 0)),
        ],
        out_specs=pl.BlockSpec(
            (1, _CH, _NBLK_L),
            lambda p, b, n: (jnp.where(p == 2, b, 0), 0,
                             jnp.where(p == 2, n, 0))),
        out_shape=jax.ShapeDtypeStruct((_B, _CH, _N), jnp.float32),
        scratch_shapes=[
            pltpu.VMEM((_ROWS, _CH), jnp.float32),
            pltpu.VMEM((8, _CH), jnp.float32),
            pltpu.VMEM((_CH, 8), jnp.float32),
        ],
    )(low_feats, g_rows, w0at, w1, g0, be0, g1, be1)


def kernel(low_theta, low_phi, low_feats, high_theta, high_phi, high_feats,
           W0, b0, g0, be0, W1, b1, g1, be1):
    del b0, b1
    w0at = W0[:, :_C1].T
    w0bt = W0[:, _C1:].T

    idx, qt = _idx_qt(low_theta, low_phi, high_theta, high_phi, high_feats, w0bt)
    g_rows = _sc_gather(qt, idx)
    return _dense(low_feats, g_rows, w0at, W1,
                  g0.reshape(1, _CH), be0.reshape(1, _CH),
                  g1.reshape(1, _CH), be1.reshape(1, _CH))


# speedup vs baseline: 3.8440x; 1.4285x over previous
import functools

import jax
import jax.numpy as jnp
from jax import lax
from jax.experimental import pallas as pl
from jax.experimental.pallas import tpu as pltpu
from jax.experimental.pallas import tpu_sc as plsc

_B, _N, _M, _C1, _C2 = 8, 4096, 1024, 128, 256
_CH = 256
_ROWS = _B * _N
_CNT = float(_ROWS)
_EPS = 1e-5


_NCHUNK_IDX = 4
_NBLK_IDX = _N // _NCHUNK_IDX


def _idx_qt_body(lt_ref, lp_ref, ht_ref, hp_ref, hf_ref, w_ref, idx_ref, qt_ref):
    b = pl.program_id(0)
    qt_ref[...] = lax.dot_general(
        hf_ref[0], w_ref[...], (((0,), (0,)), ((), ())),
        preferred_element_type=jnp.float32,
    )
    ht = jnp.transpose(ht_ref[0], (1, 0))
    hp = jnp.transpose(hp_ref[0], (1, 0))
    for c in range(_NCHUNK_IDX):
        sl = pl.ds(c * _NBLK_IDX, _NBLK_IDX)
        lt = lt_ref[0, 0:1, sl]
        lp = lp_ref[0, 0:1, sl]
        dt = ht - lt
        dp = hp - lp
        dist = dt * dt + dp * dp
        imin = jnp.argmin(dist, axis=0).astype(jnp.int32)
        idx_ref[0, 0, sl] = imin + b * _M


def _idx_qt(low_theta, low_phi, high_theta, high_phi, high_feats, w0bt):
    idx, qt = pl.pallas_call(
        _idx_qt_body,
        grid=(_B,),
        in_specs=[
            pl.BlockSpec((1, 1, _N), lambda b: (b, 0, 0)),
            pl.BlockSpec((1, 1, _N), lambda b: (b, 0, 0)),
            pl.BlockSpec((1, 1, _M), lambda b: (b, 0, 0)),
            pl.BlockSpec((1, 1, _M), lambda b: (b, 0, 0)),
            pl.BlockSpec((1, _C2, _M), lambda b: (b, 0, 0)),
            pl.BlockSpec((_C2, _CH), lambda b: (0, 0)),
        ],
        out_specs=[
            pl.BlockSpec((1, 1, _N), lambda b: (b, 0, 0)),
            pl.BlockSpec((_M, _CH), lambda b: (b, 0)),
        ],
        out_shape=[
            jax.ShapeDtypeStruct((_B, 1, _N), jnp.int32),
            jax.ShapeDtypeStruct((_B * _M, _CH), jnp.float32),
        ],
    )(low_theta.reshape(_B, 1, _N), low_phi.reshape(_B, 1, _N),
      high_theta.reshape(_B, 1, _M), high_phi.reshape(_B, 1, _M),
      high_feats, w0bt)
    return idx.reshape(_ROWS), qt


_NC = 2
_NS = 16
_NW = _NC * _NS
_RPW = _ROWS // _NW
_CHUNK = 128
_NCHUNK = _RPW // _CHUNK


def _sc_gather_body(table_hbm, idx_hbm, out_hbm, idx_v, buf0, buf1, sem0, sem1):
    wid = lax.axis_index("s") * _NC + lax.axis_index("c")
    base = wid * _RPW
    pltpu.sync_copy(idx_hbm.at[pl.ds(base, _RPW)], idx_v)
    bufs = (buf0, buf1)
    sems = (sem0, sem1)
    copies = [None, None]
    for j in range(_NCHUNK):
        p = j % 2
        if copies[p] is not None:
            copies[p].wait()
            pltpu.sync_copy(bufs[p], out_hbm.at[pl.ds(base + (j - 2) * _CHUNK, _CHUNK)])
        copies[p] = pltpu.async_copy(
            table_hbm.at[idx_v.at[pl.ds(j * _CHUNK, _CHUNK)]], bufs[p], sems[p]
        )
    for j in range(_NCHUNK - 2, _NCHUNK):
        p = j % 2
        copies[p].wait()
        pltpu.sync_copy(bufs[p], out_hbm.at[pl.ds(base + j * _CHUNK, _CHUNK)])


def _sc_gather(table, idx):
    k = functools.partial(
        pl.kernel,
        out_type=jax.ShapeDtypeStruct((_ROWS, _CH), jnp.float32),
        mesh=plsc.VectorSubcoreMesh(core_axis_name="c", subcore_axis_name="s"),
        scratch_types=[
            pltpu.VMEM((_RPW,), jnp.int32),
            pltpu.VMEM((_CHUNK, _CH), jnp.float32),
            pltpu.VMEM((_CHUNK, _CH), jnp.float32),
            pltpu.SemaphoreType.DMA,
            pltpu.SemaphoreType.DMA,
        ],
    )(_sc_gather_body)
    return k(table, idx)


_NBLK_L = 2048
_NB_L = _N // _NBLK_L


def _dense_body(lf_ref, g_ref, w0_ref, w1_ref, g0_ref, be0_ref, g1_ref, be1_ref,
                o_ref, h0_s, st0_s, st1_s):
    p = pl.program_id(0)
    b = pl.program_id(1)
    n = pl.program_id(2)
    step = b * _NB_L + n
    first = jnp.logical_and(b == 0, n == 0)
    rows = pl.ds(step * _NBLK_L, _NBLK_L)

    @pl.when(p == 0)
    def _():
        @pl.when(first)
        def _():
            st0_s[...] = jnp.zeros_like(st0_s)

        h = (
            lax.dot_general(lf_ref[0], w0_ref[...], (((0,), (0,)), ((), ())),
                            preferred_element_type=jnp.float32)
            + g_ref[...]
        )
        h0_s[rows, :] = h
        st0_s[0:1, :] += jnp.sum(h, axis=0, keepdims=True)
        st0_s[1:2, :] += jnp.sum(h * h, axis=0, keepdims=True)

    def _bn0_relu():
        mean0 = st0_s[0:1, :] * (1.0 / _CNT)
        var0 = st0_s[1:2, :] * (1.0 / _CNT) - mean0 * mean0
        scale0 = g0_ref[...] / jnp.sqrt(var0 + _EPS)
        shift0 = be0_ref[...] - scale0 * mean0
        return jnp.maximum(h0_s[rows, :] * scale0 + shift0, 0.0)

    @pl.when(p == 1)
    def _():
        @pl.when(first)
        def _():
            st1_s[...] = jnp.zeros_like(st1_s)

        h1 = lax.dot_general(w1_ref[...], _bn0_relu(), (((1,), (1,)), ((), ())),
                             preferred_element_type=jnp.float32)
        st1_s[:, 0:1] += jnp.sum(h1, axis=1, keepdims=True)
        st1_s[:, 1:2] += jnp.sum(h1 * h1, axis=1, keepdims=True)

    @pl.when(p == 2)
    def _():
        h1 = lax.dot_general(w1_ref[...], _bn0_relu(), (((1,), (1,)), ((), ())),
                             preferred_element_type=jnp.float32)
        mean1 = st1_s[:, 0:1] * (1.0 / _CNT)
        var1 = st1_s[:, 1:2] * (1.0 / _CNT) - mean1 * mean1
        g1c = jnp.transpose(g1_ref[...], (1, 0))
        be1c = jnp.transpose(be1_ref[...], (1, 0))
        scale1 = g1c / jnp.sqrt(var1 + _EPS)
        shift1 = be1c - scale1 * mean1
        o_ref[0] = jnp.maximum(h1 * scale1 + shift1, 0.0)


def _dense(low_feats, g_rows, w0at, w1, g0, be0, g1, be1):
    def _p0(i):
        return i

    return pl.pallas_call(
        _dense_body,
        grid=(3, _B, _NB_L),
        in_specs=[
            pl.BlockSpec((1, _C1, _NBLK_L),
                         lambda p, b, n: (jnp.where(p == 0, b, 0), 0,
                                          jnp.where(p == 0, n, 0))),
            pl.BlockSpec((_NBLK_L, _CH),
                         lambda p, b, n: (jnp.where(p == 0, b * _NB_L + n, 0), 0)),
            pl.BlockSpec((_C1, _CH), lambda p, b, n: (0, 0)),
            pl.BlockSpec((_CH, _CH), lambda p, b, n: (0, 0)),
            pl.BlockSpec((1, _CH), lambda p, b, n: (0, 0)),
            pl.BlockSpec((1, _CH), lambda p, b, n: (0, 0)),
            pl.BlockSpec((1, _CH), lambda p, b, n: (0, 0)),
            pl.BlockSpec((1, _CH), lambda p, b, n: (0, 0)),
        ],
        out_specs=pl.BlockSpec(
            (1, _CH, _NBLK_L),
            lambda p, b, n: (jnp.where(p == 2, b, 0), 0,
                             jnp.where(p == 2, n, 0))),
        out_shape=jax.ShapeDtypeStruct((_B, _CH, _N), jnp.float32),
        scratch_shapes=[
            pltpu.VMEM((_ROWS, _CH), jnp.float32),
            pltpu.VMEM((8, _CH), jnp.float32),
            pltpu.VMEM((_CH, 8), jnp.float32),
        ],
    )(low_feats, g_rows, w0at, w1, g0, be0, g1, be1)


def kernel(low_theta, low_phi, low_feats, high_theta, high_phi, high_feats,
           W0, b0, g0, be0, W1, b1, g1, be1):
    del b0, b1
    w0at = W0[:, :_C1].T
    w0bt = W0[:, _C1:].T

    idx, qt = _idx_qt(low_theta, low_phi, high_theta, high_phi, high_feats, w0bt)
    g_rows = _sc_gather(qt, idx)
    return _dense(low_feats, g_rows, w0at, W1,
                  g0.reshape(1, _CH), be0.reshape(1, _CH),
                  g1.reshape(1, _CH), be1.reshape(1, _CH))


# speedup vs baseline: 4.4152x; 1.1486x over previous
import functools

import jax
import jax.numpy as jnp
from jax import lax
from jax.experimental import pallas as pl
from jax.experimental.pallas import tpu as pltpu
from jax.experimental.pallas import tpu_sc as plsc

_B, _N, _M, _C1, _C2 = 8, 4096, 1024, 128, 256
_CH = 256
_ROWS = _B * _N
_CNT = float(_ROWS)
_EPS = 1e-5


_NCHUNK_IDX = 4
_NBLK_IDX = _N // _NCHUNK_IDX


def _idx_qt_body(lt_ref, lp_ref, ht_ref, hp_ref, hf_ref, w_ref, idx_ref, qt_ref):
    b = pl.program_id(0)
    qt_ref[...] = lax.dot_general(
        hf_ref[0], w_ref[...], (((0,), (0,)), ((), ())),
        preferred_element_type=jnp.float32,
    )
    ht = jnp.transpose(ht_ref[0], (1, 0))
    hp = jnp.transpose(hp_ref[0], (1, 0))
    riota = lax.broadcasted_iota(jnp.int32, (8, _NBLK_IDX), 0)
    for c in range(_NCHUNK_IDX):
        sl = pl.ds(c * _NBLK_IDX, _NBLK_IDX)
        lt = lt_ref[0, 0:1, sl]
        lp = lp_ref[0, 0:1, sl]
        minv = jnp.full((8, _NBLK_IDX), jnp.inf, jnp.float32)
        mini = jnp.zeros((8, _NBLK_IDX), jnp.int32)
        for s in range(_M // 8):
            dt = ht[s * 8:(s + 1) * 8, :] - lt
            dp = hp[s * 8:(s + 1) * 8, :] - lp
            d2 = dt * dt + dp * dp
            cond = d2 < minv
            minv = jnp.where(cond, d2, minv)
            mini = jnp.where(cond, riota + (8 * s), mini)
        gmin = jnp.min(minv, axis=0, keepdims=True)
        cand = jnp.where(minv == gmin, mini, _M)
        imin = jnp.min(cand, axis=0).astype(jnp.int32)
        idx_ref[0, 0, sl] = imin + b * _M


def _idx_qt(low_theta, low_phi, high_theta, high_phi, high_feats, w0bt):
    idx, qt = pl.pallas_call(
        _idx_qt_body,
        grid=(_B,),
        in_specs=[
            pl.BlockSpec((1, 1, _N), lambda b: (b, 0, 0)),
            pl.BlockSpec((1, 1, _N), lambda b: (b, 0, 0)),
            pl.BlockSpec((1, 1, _M), lambda b: (b, 0, 0)),
            pl.BlockSpec((1, 1, _M), lambda b: (b, 0, 0)),
            pl.BlockSpec((1, _C2, _M), lambda b: (b, 0, 0)),
            pl.BlockSpec((_C2, _CH), lambda b: (0, 0)),
        ],
        out_specs=[
            pl.BlockSpec((1, 1, _N), lambda b: (b, 0, 0)),
            pl.BlockSpec((_M, _CH), lambda b: (b, 0)),
        ],
        out_shape=[
            jax.ShapeDtypeStruct((_B, 1, _N), jnp.int32),
            jax.ShapeDtypeStruct((_B * _M, _CH), jnp.float32),
        ],
    )(low_theta.reshape(_B, 1, _N), low_phi.reshape(_B, 1, _N),
      high_theta.reshape(_B, 1, _M), high_phi.reshape(_B, 1, _M),
      high_feats, w0bt)
    return idx.reshape(_ROWS), qt


_NC = 2
_NS = 16
_NW = _NC * _NS
_RPW = _ROWS // _NW
_CHUNK = 128
_NCHUNK = _RPW // _CHUNK


def _sc_gather_body(table_hbm, idx_hbm, out_hbm, idx_v, buf0, buf1, sem0, sem1):
    wid = lax.axis_index("s") * _NC + lax.axis_index("c")
    base = wid * _RPW
    pltpu.sync_copy(idx_hbm.at[pl.ds(base, _RPW)], idx_v)
    bufs = (buf0, buf1)
    sems = (sem0, sem1)
    copies = [None, None]
    for j in range(_NCHUNK):
        p = j % 2
        if copies[p] is not None:
            copies[p].wait()
            pltpu.sync_copy(bufs[p], out_hbm.at[pl.ds(base + (j - 2) * _CHUNK, _CHUNK)])
        copies[p] = pltpu.async_copy(
            table_hbm.at[idx_v.at[pl.ds(j * _CHUNK, _CHUNK)]], bufs[p], sems[p]
        )
    for j in range(_NCHUNK - 2, _NCHUNK):
        p = j % 2
        copies[p].wait()
        pltpu.sync_copy(bufs[p], out_hbm.at[pl.ds(base + j * _CHUNK, _CHUNK)])


def _sc_gather(table, idx):
    k = functools.partial(
        pl.kernel,
        out_type=jax.ShapeDtypeStruct((_ROWS, _CH), jnp.float32),
        mesh=plsc.VectorSubcoreMesh(core_axis_name="c", subcore_axis_name="s"),
        scratch_types=[
            pltpu.VMEM((_RPW,), jnp.int32),
            pltpu.VMEM((_CHUNK, _CH), jnp.float32),
            pltpu.VMEM((_CHUNK, _CH), jnp.float32),
            pltpu.SemaphoreType.DMA,
            pltpu.SemaphoreType.DMA,
        ],
    )(_sc_gather_body)
    return k(table, idx)


_NBLK_L = 2048
_NB_L = _N // _NBLK_L


def _dense_body(lf_ref, g_ref, w0_ref, w1_ref, g0_ref, be0_ref, g1_ref, be1_ref,
                o_ref, h0_s, st0_s, st1_s):
    p = pl.program_id(0)
    b = pl.program_id(1)
    n = pl.program_id(2)
    step = b * _NB_L + n
    first = jnp.logical_and(b == 0, n == 0)
    rows = pl.ds(step * _NBLK_L, _NBLK_L)

    @pl.when(p == 0)
    def _():
        @pl.when(first)
        def _():
            st0_s[...] = jnp.zeros_like(st0_s)

        h = (
            lax.dot_general(lf_ref[0], w0_ref[...], (((0,), (0,)), ((), ())),
                            preferred_element_type=jnp.float32)
            + g_ref[...]
        )
        h0_s[rows, :] = h
        st0_s[0:1, :] += jnp.sum(h, axis=0, keepdims=True)
        st0_s[1:2, :] += jnp.sum(h * h, axis=0, keepdims=True)

    def _bn0_relu():
        mean0 = st0_s[0:1, :] * (1.0 / _CNT)
        var0 = st0_s[1:2, :] * (1.0 / _CNT) - mean0 * mean0
        scale0 = g0_ref[...] / jnp.sqrt(var0 + _EPS)
        shift0 = be0_ref[...] - scale0 * mean0
        return jnp.maximum(h0_s[rows, :] * scale0 + shift0, 0.0)

    @pl.when(p == 1)
    def _():
        @pl.when(first)
        def _():
            st1_s[...] = jnp.zeros_like(st1_s)

        h1 = lax.dot_general(w1_ref[...], _bn0_relu(), (((1,), (1,)), ((), ())),
                             preferred_element_type=jnp.float32)
        st1_s[:, 0:1] += jnp.sum(h1, axis=1, keepdims=True)
        st1_s[:, 1:2] += jnp.sum(h1 * h1, axis=1, keepdims=True)

    @pl.when(p == 2)
    def _():
        h1 = lax.dot_general(w1_ref[...], _bn0_relu(), (((1,), (1,)), ((), ())),
                             preferred_element_type=jnp.float32)
        mean1 = st1_s[:, 0:1] * (1.0 / _CNT)
        var1 = st1_s[:, 1:2] * (1.0 / _CNT) - mean1 * mean1
        g1c = jnp.transpose(g1_ref[...], (1, 0))
        be1c = jnp.transpose(be1_ref[...], (1, 0))
        scale1 = g1c / jnp.sqrt(var1 + _EPS)
        shift1 = be1c - scale1 * mean1
        o_ref[0] = jnp.maximum(h1 * scale1 + shift1, 0.0)


def _dense(low_feats, g_rows, w0at, w1, g0, be0, g1, be1):
    def _p0(i):
        return i

    return pl.pallas_call(
        _dense_body,
        grid=(3, _B, _NB_L),
        in_specs=[
            pl.BlockSpec((1, _C1, _NBLK_L),
                         lambda p, b, n: (jnp.where(p == 0, b, 0), 0,
                                          jnp.where(p == 0, n, 0))),
            pl.BlockSpec((_NBLK_L, _CH),
                         lambda p, b, n: (jnp.where(p == 0, b * _NB_L + n, 0), 0)),
            pl.BlockSpec((_C1, _CH), lambda p, b, n: (0, 0)),
            pl.BlockSpec((_CH, _CH), lambda p, b, n: (0, 0)),
            pl.BlockSpec((1, _CH), lambda p, b, n: (0, 0)),
            pl.BlockSpec((1, _CH), lambda p, b, n: (0, 0)),
            pl.BlockSpec((1, _CH), lambda p, b, n: (0, 0)),
            pl.BlockSpec((1, _CH), lambda p, b, n: (0, 0)),
        ],
        out_specs=pl.BlockSpec(
            (1, _CH, _NBLK_L),
            lambda p, b, n: (jnp.where(p == 2, b, 0), 0,
                             jnp.where(p == 2, n, 0))),
        out_shape=jax.ShapeDtypeStruct((_B, _CH, _N), jnp.float32),
        scratch_shapes=[
            pltpu.VMEM((_ROWS, _CH), jnp.float32),
            pltpu.VMEM((8, _CH), jnp.float32),
            pltpu.VMEM((_CH, 8), jnp.float32),
        ],
    )(low_feats, g_rows, w0at, w1, g0, be0, g1, be1)


def kernel(low_theta, low_phi, low_feats, high_theta, high_phi, high_feats,
           W0, b0, g0, be0, W1, b1, g1, be1):
    del b0, b1
    w0at = W0[:, :_C1].T
    w0bt = W0[:, _C1:].T

    idx, qt = _idx_qt(low_theta, low_phi, high_theta, high_phi, high_feats, w0bt)
    g_rows = _sc_gather(qt, idx)
    return _dense(low_feats, g_rows, w0at, W1,
                  g0.reshape(1, _CH), be0.reshape(1, _CH),
                  g1.reshape(1, _CH), be1.reshape(1, _CH))


# speedup vs baseline: 4.7189x; 1.0688x over previous
import functools

import jax
import jax.numpy as jnp
from jax import lax
from jax.experimental import pallas as pl
from jax.experimental.pallas import tpu as pltpu
from jax.experimental.pallas import tpu_sc as plsc

_B, _N, _M, _C1, _C2 = 8, 4096, 1024, 128, 256
_CH = 256
_ROWS = _B * _N
_CNT = float(_ROWS)
_EPS = 1e-5


_NCHUNK_IDX = 4
_NBLK_IDX = _N // _NCHUNK_IDX


def _idx_qt_body(lt_ref, lp_ref, ht_ref, hp_ref, hf_ref, w_ref, idx_ref, qt_ref):
    b = pl.program_id(0)
    qt_ref[...] = lax.dot_general(
        hf_ref[0], w_ref[...], (((0,), (0,)), ((), ())),
        preferred_element_type=jnp.float32,
    )
    ht = jnp.transpose(ht_ref[0], (1, 0))
    hp = jnp.transpose(hp_ref[0], (1, 0))
    riota = lax.broadcasted_iota(jnp.int32, (8, _NBLK_IDX), 0)
    for c in range(_NCHUNK_IDX):
        sl = pl.ds(c * _NBLK_IDX, _NBLK_IDX)
        lt = lt_ref[0, 0:1, sl]
        lp = lp_ref[0, 0:1, sl]
        minv = jnp.full((8, _NBLK_IDX), jnp.inf, jnp.float32)
        mini = jnp.zeros((8, _NBLK_IDX), jnp.int32)
        for s in range(_M // 8):
            dt = ht[s * 8:(s + 1) * 8, :] - lt
            dp = hp[s * 8:(s + 1) * 8, :] - lp
            d2 = dt * dt + dp * dp
            cond = d2 < minv
            minv = jnp.where(cond, d2, minv)
            mini = jnp.where(cond, riota + (8 * s), mini)
        gmin = jnp.min(minv, axis=0, keepdims=True)
        cand = jnp.where(minv == gmin, mini, _M)
        imin = jnp.min(cand, axis=0).astype(jnp.int32)
        idx_ref[0, 0, sl] = imin + b * _M


def _idx_qt(low_theta, low_phi, high_theta, high_phi, high_feats, w0bt):
    idx, qt = pl.pallas_call(
        _idx_qt_body,
        grid=(_B,),
        in_specs=[
            pl.BlockSpec((1, 1, _N), lambda b: (b, 0, 0)),
            pl.BlockSpec((1, 1, _N), lambda b: (b, 0, 0)),
            pl.BlockSpec((1, 1, _M), lambda b: (b, 0, 0)),
            pl.BlockSpec((1, 1, _M), lambda b: (b, 0, 0)),
            pl.BlockSpec((1, _C2, _M), lambda b: (b, 0, 0)),
            pl.BlockSpec((_C2, _CH), lambda b: (0, 0)),
        ],
        out_specs=[
            pl.BlockSpec((1, 1, _N), lambda b: (b, 0, 0)),
            pl.BlockSpec((_M, _CH), lambda b: (b, 0)),
        ],
        out_shape=[
            jax.ShapeDtypeStruct((_B, 1, _N), jnp.int32),
            jax.ShapeDtypeStruct((_B * _M, _CH), jnp.float32),
        ],
    )(low_theta.reshape(_B, 1, _N), low_phi.reshape(_B, 1, _N),
      high_theta.reshape(_B, 1, _M), high_phi.reshape(_B, 1, _M),
      high_feats, w0bt)
    return idx.reshape(_ROWS), qt


_NC = 2
_NS = 16
_NW = _NC * _NS
_RPW = _ROWS // _NW
_CHUNK = 128
_NCHUNK = _RPW // _CHUNK


def _sc_gather_body(table_hbm, idx_hbm, out_hbm, idx_v, buf0, buf1, sem0, sem1):
    wid = lax.axis_index("s") * _NC + lax.axis_index("c")
    base = wid * _RPW
    pltpu.sync_copy(idx_hbm.at[pl.ds(base, _RPW)], idx_v)
    bufs = (buf0, buf1)
    sems = (sem0, sem1)
    copies = [None, None]
    for j in range(_NCHUNK):
        p = j % 2
        if copies[p] is not None:
            copies[p].wait()
            pltpu.sync_copy(bufs[p], out_hbm.at[pl.ds(base + (j - 2) * _CHUNK, _CHUNK)])
        copies[p] = pltpu.async_copy(
            table_hbm.at[idx_v.at[pl.ds(j * _CHUNK, _CHUNK)]], bufs[p], sems[p]
        )
    for j in range(_NCHUNK - 2, _NCHUNK):
        p = j % 2
        copies[p].wait()
        pltpu.sync_copy(bufs[p], out_hbm.at[pl.ds(base + j * _CHUNK, _CHUNK)])


def _sc_gather(table, idx):
    k = functools.partial(
        pl.kernel,
        out_type=jax.ShapeDtypeStruct((_ROWS, _CH), jnp.float32),
        mesh=plsc.VectorSubcoreMesh(core_axis_name="c", subcore_axis_name="s"),
        scratch_types=[
            pltpu.VMEM((_RPW,), jnp.int32),
            pltpu.VMEM((_CHUNK, _CH), jnp.float32),
            pltpu.VMEM((_CHUNK, _CH), jnp.float32),
            pltpu.SemaphoreType.DMA,
            pltpu.SemaphoreType.DMA,
        ],
    )(_sc_gather_body)
    return k(table, idx)


_NBLK_L = 4096
_NB_L = _N // _NBLK_L


def _dense_body(lf_ref, g_ref, w0_ref, w1_ref, g0_ref, be0_ref, g1_ref, be1_ref,
                o_ref, h0_s, st0_s, st1_s):
    p = pl.program_id(0)
    b = pl.program_id(1)
    n = pl.program_id(2)
    step = b * _NB_L + n
    first = jnp.logical_and(b == 0, n == 0)
    rows = pl.ds(step * _NBLK_L, _NBLK_L)

    @pl.when(p == 0)
    def _():
        @pl.when(first)
        def _():
            st0_s[...] = jnp.zeros_like(st0_s)

        h = (
            lax.dot_general(lf_ref[0], w0_ref[...], (((0,), (0,)), ((), ())),
                            preferred_element_type=jnp.float32)
            + g_ref[...]
        )
        h0_s[rows, :] = h
        st0_s[0:1, :] += jnp.sum(h, axis=0, keepdims=True)
        st0_s[1:2, :] += jnp.sum(h * h, axis=0, keepdims=True)

    def _bn0_relu():
        mean0 = st0_s[0:1, :] * (1.0 / _CNT)
        var0 = st0_s[1:2, :] * (1.0 / _CNT) - mean0 * mean0
        scale0 = g0_ref[...] / jnp.sqrt(var0 + _EPS)
        shift0 = be0_ref[...] - scale0 * mean0
        return jnp.maximum(h0_s[rows, :] * scale0 + shift0, 0.0)

    @pl.when(p == 1)
    def _():
        @pl.when(first)
        def _():
            st1_s[...] = jnp.zeros_like(st1_s)

        h1 = lax.dot_general(w1_ref[...], _bn0_relu(), (((1,), (1,)), ((), ())),
                             preferred_element_type=jnp.float32)
        st1_s[:, 0:1] += jnp.sum(h1, axis=1, keepdims=True)
        st1_s[:, 1:2] += jnp.sum(h1 * h1, axis=1, keepdims=True)

    @pl.when(p == 2)
    def _():
        h1 = lax.dot_general(w1_ref[...], _bn0_relu(), (((1,), (1,)), ((), ())),
                             preferred_element_type=jnp.float32)
        mean1 = st1_s[:, 0:1] * (1.0 / _CNT)
        var1 = st1_s[:, 1:2] * (1.0 / _CNT) - mean1 * mean1
        g1c = jnp.transpose(g1_ref[...], (1, 0))
        be1c = jnp.transpose(be1_ref[...], (1, 0))
        scale1 = g1c / jnp.sqrt(var1 + _EPS)
        shift1 = be1c - scale1 * mean1
        o_ref[0] = jnp.maximum(h1 * scale1 + shift1, 0.0)


def _dense(low_feats, g_rows, w0at, w1, g0, be0, g1, be1):
    def _p0(i):
        return i

    return pl.pallas_call(
        _dense_body,
        grid=(3, _B, _NB_L),
        in_specs=[
            pl.BlockSpec((1, _C1, _NBLK_L),
                         lambda p, b, n: (jnp.where(p == 0, b, 0), 0,
                                          jnp.where(p == 0, n, 0))),
            pl.BlockSpec((_NBLK_L, _CH),
                         lambda p, b, n: (jnp.where(p == 0, b * _NB_L + n, 0), 0)),
            pl.BlockSpec((_C1, _CH), lambda p, b, n: (0, 0)),
            pl.BlockSpec((_CH, _CH), lambda p, b, n: (0, 0)),
            pl.BlockSpec((1, _CH), lambda p, b, n: (0, 0)),
            pl.BlockSpec((1, _CH), lambda p, b, n: (0, 0)),
            pl.BlockSpec((1, _CH), lambda p, b, n: (0, 0)),
            pl.BlockSpec((1, _CH), lambda p, b, n: (0, 0)),
        ],
        out_specs=pl.BlockSpec(
            (1, _CH, _NBLK_L),
            lambda p, b, n: (jnp.where(p == 2, b, 0), 0,
                             jnp.where(p == 2, n, 0))),
        out_shape=jax.ShapeDtypeStruct((_B, _CH, _N), jnp.float32),
        scratch_shapes=[
            pltpu.VMEM((_ROWS, _CH), jnp.float32),
            pltpu.VMEM((8, _CH), jnp.float32),
            pltpu.VMEM((_CH, 8), jnp.float32),
        ],
    )(low_feats, g_rows, w0at, w1, g0, be0, g1, be1)


def kernel(low_theta, low_phi, low_feats, high_theta, high_phi, high_feats,
           W0, b0, g0, be0, W1, b1, g1, be1):
    del b0, b1
    w0at = W0[:, :_C1].T
    w0bt = W0[:, _C1:].T

    idx, qt = _idx_qt(low_theta, low_phi, high_theta, high_phi, high_feats, w0bt)
    g_rows = _sc_gather(qt, idx)
    return _dense(low_feats, g_rows, w0at, W1,
                  g0.reshape(1, _CH), be0.reshape(1, _CH),
                  g1.reshape(1, _CH), be1.reshape(1, _CH))
